# Initial kernel scaffold; baseline (speedup 1.0000x reference)
import jax
import jax.numpy as jnp
from jax.experimental import pallas as pl


def kernel(x, edge_index, W1, b1, W2, b2):
    raise NotImplementedError("write your pallas kernel here")


# speedup vs baseline: 10.7193x; 10.7193x over previous
import jax
import jax.numpy as jnp
from jax import lax
from jax.experimental import pallas as pl
from jax.experimental.pallas import tpu as pltpu
from jax.experimental.pallas import tpu_sc as plsc

_EPS = 1e-12
_K = 128
_NW = 32


def _l2n(t):
    return t / (jnp.sqrt(jnp.sum(t * t)) + _EPS)


def _sigma(w):
    n_in = w.shape[0]
    u0 = jnp.full((1, n_in), 1.0, jnp.float32) / (
        jnp.sqrt(jnp.float32(n_in)) + _EPS)
    v = _l2n(jnp.dot(u0, w, preferred_element_type=jnp.float32))
    u1 = _l2n(lax.dot_general(v, w, (((1,), (1,)), ((), ())),
                              preferred_element_type=jnp.float32))
    return jnp.sum(jnp.dot(u1, w, preferred_element_type=jnp.float32) * v)


def _tc1_body(n):
    def body(x_ref, w_ref, degp_ref, y_ref, dinv_ref):
        deg = degp_ref[0, 0:n, 0:1] + degp_ref[1, 0:n, 0:1] + 1.0
        dinv = jnp.where(deg > 0, 1.0 / jnp.sqrt(deg), 0.0)
        w = w_ref[...]
        sig = _sigma(w)
        xw = jnp.dot(x_ref[...], w, preferred_element_type=jnp.float32)
        y_ref[...] = (dinv / sig) * xw
        dinv_ref[...] = dinv
    return body


def _tc2_body(n):
    def body(sp_ref, y_ref, dinv_ref, b_ref, w_ref, h_ref, y2_ref):
        dinv = dinv_ref[...]
        ssum = sp_ref[0, 0:n, :] + sp_ref[1, 0:n, :]
        h = jnp.maximum(dinv * (ssum + y_ref[...]) + b_ref[...], 0.0)
        w = w_ref[...]
        sig = _sigma(w)
        y2_ref[...] = (dinv / sig) * jnp.dot(
            h, w, preferred_element_type=jnp.float32)
        h_ref[...] = h
    return body


def _tc3_body(n):
    def body(sp_ref, y2_ref, h1_ref, dinv_ref, b_ref, out_ref):
        dinv = dinv_ref[...]
        ssum = sp_ref[0, 0:n, :] + sp_ref[1, 0:n, :]
        h2 = jnp.maximum(dinv * (ssum + y2_ref[...]) + b_ref[...], 0.0)
        out_ref[...] = jnp.maximum(h1_ref[...], h2)
    return body


def _make_deg(e_pad, n_pad):
    ept = e_pad // _NW
    nck = ept // _K
    rpt = n_pad // 16
    ncopy = rpt // _K

    def body(col_hbm, out_hbm, acc, col_v, ones_v, zbuf):
        c = lax.axis_index("c")
        s = lax.axis_index("s")
        wid = s * 2 + c
        zero16 = jnp.zeros((16,), jnp.float32)
        one16 = jnp.ones((16,), jnp.float32)

        def zfill(i, _):
            for j in range(8):
                zbuf[i, pl.ds(j * 16, 16)] = zero16
                ones_v[i, pl.ds(j * 16, 16)] = one16
            return 0
        lax.fori_loop(0, _K, zfill, 0)
        for t in range(ncopy):
            pltpu.sync_copy(zbuf, acc.at[pl.ds(s * rpt + t * _K, _K)])
        plsc.subcore_barrier()

        base0 = wid * ept

        def step(g, _):
            pltpu.sync_copy(col_hbm.at[pl.ds(base0 + g * _K, _K)], col_v)
            pltpu.sync_copy(ones_v, acc.at[col_v], add=True)
            return 0
        lax.fori_loop(0, nck, step, 0)

        plsc.subcore_barrier()
        for t in range(ncopy):
            off = s * rpt + t * _K
            pltpu.sync_copy(acc.at[pl.ds(off, _K)],
                            out_hbm.at[c, pl.ds(off, _K)])

    return pl.kernel(
        body,
        mesh=plsc.VectorSubcoreMesh(core_axis_name="c", subcore_axis_name="s"),
        out_type=jax.ShapeDtypeStruct((2, n_pad, 128), jnp.float32),
        scratch_types=[
            pltpu.VMEM_SHARED((n_pad, 128), jnp.float32),
            pltpu.VMEM((_K,), jnp.int32),
            pltpu.VMEM((_K, 128), jnp.float32),
            pltpu.VMEM((_K, 128), jnp.float32),
        ],
    )


def _make_edge(e_pad, n_pad):
    ept = e_pad // _NW
    nck = ept // _K
    rpt = n_pad // 16
    ncopy = rpt // _K

    def body(y_hbm, row_hbm, col_hbm, out_hbm,
             acc, row_v, col_v, rows_v, zbuf, sem):
        c = lax.axis_index("c")
        s = lax.axis_index("s")
        wid = s * 2 + c
        zero16 = jnp.zeros((16,), jnp.float32)

        def zfill(i, _):
            for j in range(8):
                zbuf[i, pl.ds(j * 16, 16)] = zero16
            return 0
        lax.fori_loop(0, _K, zfill, 0)
        for t in range(ncopy):
            pltpu.sync_copy(zbuf, acc.at[pl.ds(s * rpt + t * _K, _K)])
        plsc.subcore_barrier()

        base0 = wid * ept

        def step(g, _):
            b = base0 + g * _K
            pltpu.sync_copy(row_hbm.at[pl.ds(b, _K)], row_v)
            pltpu.sync_copy(col_hbm.at[pl.ds(b, _K)], col_v)
            pltpu.async_copy(y_hbm.at[row_v], rows_v, sem).wait()
            pltpu.sync_copy(rows_v, acc.at[col_v], add=True)
            return 0
        lax.fori_loop(0, nck, step, 0)

        plsc.subcore_barrier()
        for t in range(ncopy):
            off = s * rpt + t * _K
            pltpu.sync_copy(acc.at[pl.ds(off, _K)],
                            out_hbm.at[c, pl.ds(off, _K)])

    return pl.kernel(
        body,
        mesh=plsc.VectorSubcoreMesh(core_axis_name="c", subcore_axis_name="s"),
        out_type=jax.ShapeDtypeStruct((2, n_pad, 128), jnp.float32),
        scratch_types=[
            pltpu.VMEM_SHARED((n_pad, 128), jnp.float32),
            pltpu.VMEM((_K,), jnp.int32),
            pltpu.VMEM((_K,), jnp.int32),
            pltpu.VMEM((_K, 128), jnp.float32),
            pltpu.VMEM((_K, 128), jnp.float32),
            pltpu.SemaphoreType.DMA,
        ],
    )


def kernel(x, edge_index, W1, b1, W2, b2):
    n, d = x.shape
    e = edge_index.shape[1]
    h = W1.shape[1]

    chunk_all = _NW * _K
    e_pad = ((e + chunk_all - 1) // chunk_all) * chunk_all
    n_pad = ((n + 1 + 2047) // 2048) * 2048

    row = edge_index[0]
    col = edge_index[1]
    pad = e_pad - e
    row_p = jnp.concatenate([row, jnp.zeros((pad,), row.dtype)])
    col_p = jnp.concatenate([col, jnp.full((pad,), n, col.dtype)])

    degp = _make_deg(e_pad, n_pad)(col_p)

    y1, dinv = pl.pallas_call(
        _tc1_body(n),
        out_shape=[jax.ShapeDtypeStruct((n, h), jnp.float32),
                   jax.ShapeDtypeStruct((n, 1), jnp.float32)],
    )(x, W1, degp)

    edge_pass = _make_edge(e_pad, n_pad)
    s1p = edge_pass(y1, row_p, col_p)

    h1, y2 = pl.pallas_call(
        _tc2_body(n),
        out_shape=[jax.ShapeDtypeStruct((n, h), jnp.float32),
                   jax.ShapeDtypeStruct((n, h), jnp.float32)],
    )(s1p, y1, dinv, b1.reshape(1, -1), W2)

    s2p = edge_pass(y2, row_p, col_p)

    out = pl.pallas_call(
        _tc3_body(n),
        out_shape=jax.ShapeDtypeStruct((n, h), jnp.float32),
    )(s2p, y2, h1, dinv, b2.reshape(1, -1))
    return out
